# baseline (speedup 1.0000x reference)
import functools

import jax
import jax.numpy as jnp
from jax import lax
from jax.experimental import pallas as pl
from jax.experimental.pallas import tpu as pltpu
from jax.experimental.pallas import tpu_sc as plsc

_DIM = 1024
_E = 8
_H = 2560
_N = 8192
_T = 256
_NBT = _N // _T
_BLK = 256
_NB = (2 * _N) // _BLK + _E
_PADT = _NB * _BLK
_CH = 512
_NW = 32
_F32 = jnp.float32


def _gate_body(x_ref, gw_ref, d0_ref, d1_ref, w0_ref, w1_ref, be_ref,
               aux_ref, acc_ref, imp_ref, ident_ref, ltri_ref):
    ph = pl.program_id(0)
    b = pl.program_id(1)

    @pl.when((ph == 0) & (b == 0))
    def _init():
        acc_ref[...] = jnp.zeros_like(acc_ref)
        imp_ref[...] = jnp.zeros_like(imp_ref)
        ident_ref[...] = (
            lax.broadcasted_iota(jnp.int32, (_T, _T), 0)
            == lax.broadcasted_iota(jnp.int32, (_T, _T), 1)).astype(_F32)
        ltri_ref[...] = (
            lax.broadcasted_iota(jnp.int32, (2 * _T, 2 * _T), 0)
            < lax.broadcasted_iota(jnp.int32, (2 * _T, 2 * _T), 1)).astype(_F32)

    x = x_ref[...]
    gw = gw_ref[...]
    logits = lax.dot_general(x, gw, (((1,), (1,)), ((), ())),
                             preferred_element_type=_F32)
    m = jnp.max(logits, axis=1, keepdims=True)
    ex = jnp.exp(logits - m)
    probs = ex / jnp.sum(ex, axis=1, keepdims=True)

    ei = lax.broadcasted_iota(jnp.int32, (_T, _E), 1).astype(_F32)
    m1 = jnp.max(probs, axis=1, keepdims=True)
    i1 = jnp.min(jnp.where(probs == m1, ei, 8.0), axis=1, keepdims=True)
    oh1 = (ei == i1).astype(_F32)
    pm = probs - 2.0 * oh1
    m2 = jnp.max(pm, axis=1, keepdims=True)
    i2 = jnp.min(jnp.where(pm == m2, ei, 8.0), axis=1, keepdims=True)

    icat = jnp.concatenate([i1, i2], axis=1)
    it2 = lax.dot_general(icat, ident_ref[...], (((0,), (0,)), ((), ())),
                          preferred_element_type=_F32)
    e8 = lax.broadcasted_iota(jnp.int32, (_E, 2 * _T), 0).astype(_F32)
    ipairs = jnp.concatenate([it2[0:1, :], it2[1:2, :]], axis=1)
    oht = (e8 == ipairs).astype(_F32)

    @pl.when(ph == 0)
    def _count():
        acc_ref[:, 0:1] += jnp.sum(oht, axis=1, keepdims=True)
        imp_ref[0:1, 0:8] += jnp.sum(probs, axis=0, keepdims=True)

    @pl.when((ph == 0) & (b == _NBT - 1))
    def _offsets():
        cnt = acc_ref[:, 0:1]
        nb = jnp.floor((cnt + (_BLK - 1.0)) * (1.0 / _BLK))
        tril8 = (lax.broadcasted_iota(jnp.int32, (_E, _E), 0)
                 > lax.broadcasted_iota(jnp.int32, (_E, _E), 1)).astype(_F32)
        offs = lax.dot_general(tril8, nb, (((1,), (0,)), ((), ())),
                               preferred_element_type=_F32)
        acc_ref[:, 2:3] = offs * float(_BLK)
        acc_ref[:, 1:2] = jnp.zeros((_E, 1), _F32)
        acc_ref[:, 3:4] = offs + nb

    @pl.when(ph == 1)
    def _rank():
        ranks = lax.dot_general(oht, ltri_ref[...], (((1,), (0,)), ((), ())),
                                preferred_element_type=_F32)
        rank = jnp.sum(oht * ranks, axis=0, keepdims=True)
        basecol = acc_ref[:, 2:3] + acc_ref[:, 1:2]
        base = jnp.sum(oht * basecol, axis=0, keepdims=True)
        dest = base + rank
        wcat = jnp.concatenate([m1 / (m1 + m2), m2 / (m1 + m2)], axis=1)
        wt2 = lax.dot_general(wcat, ident_ref[...], (((0,), (0,)), ((), ())),
                              preferred_element_type=_F32,
                              precision=lax.Precision.HIGHEST)
        d0_ref[...] = dest[:, 0:_T].astype(jnp.int32).reshape(1, 1, _T)
        d1_ref[...] = dest[:, _T:2 * _T].astype(jnp.int32).reshape(1, 1, _T)
        w0_ref[...] = wt2[0:1, :].reshape(1, 1, _T)
        w1_ref[...] = wt2[1:2, :].reshape(1, 1, _T)
        acc_ref[:, 1:2] += jnp.sum(oht, axis=1, keepdims=True)

    @pl.when((ph == 1) & (b == _NBT - 1))
    def _finish():
        aux = lax.dot_general(imp_ref[0:1, 0:8], acc_ref[:, 0:1],
                              (((1,), (0,)), ((), ())),
                              preferred_element_type=_F32,
                              precision=lax.Precision.HIGHEST)
        aux_ref[...] = aux * (float(_E) / (float(_N) * float(_N)))
        jv = lax.broadcasted_iota(jnp.int32, (1, 128), 1).astype(_F32)
        bx = jnp.sum((acc_ref[:, 3:4] <= jv).astype(_F32), axis=0,
                     keepdims=True)
        used = jnp.sum(acc_ref[7:8, 3:4])
        bxx = jnp.where(jv >= 127.0, used, jnp.minimum(bx, 7.0))
        be_ref[...] = bxx.astype(jnp.int32)


def _gate_call(xf, gate_w):
    return pl.pallas_call(
        _gate_body,
        grid=(2, _NBT),
        in_specs=[
            pl.BlockSpec((_T, _DIM), lambda p, b: (b, 0)),
            pl.BlockSpec((_E, _DIM), lambda p, b: (0, 0)),
        ],
        out_specs=[
            pl.BlockSpec((1, 1, _T), lambda p, b: (b, 0, 0)),
            pl.BlockSpec((1, 1, _T), lambda p, b: (b, 0, 0)),
            pl.BlockSpec((1, 1, _T), lambda p, b: (b, 0, 0)),
            pl.BlockSpec((1, 1, _T), lambda p, b: (b, 0, 0)),
            pl.BlockSpec((1, 128), lambda p, b: (0, 0)),
            pl.BlockSpec((1, 1), lambda p, b: (0, 0)),
        ],
        out_shape=[
            jax.ShapeDtypeStruct((_NBT, 1, _T), jnp.int32),
            jax.ShapeDtypeStruct((_NBT, 1, _T), jnp.int32),
            jax.ShapeDtypeStruct((_NBT, 1, _T), _F32),
            jax.ShapeDtypeStruct((_NBT, 1, _T), _F32),
            jax.ShapeDtypeStruct((1, 128), jnp.int32),
            jax.ShapeDtypeStruct((1, 1), _F32),
        ],
        scratch_shapes=[pltpu.VMEM((_E, 128), _F32),
                        pltpu.VMEM((_E, 128), _F32),
                        pltpu.VMEM((_T, _T), _F32),
                        pltpu.VMEM((2 * _T, 2 * _T), _F32)],
    )(xf, gate_w)


def _scatter_body(xf_hbm, d0_hbm, d1_hbm, xg_hbm, idx_v, rows_v, sem):
    wid = lax.axis_index("s") * 2 + lax.axis_index("c")
    per = _N // _NW
    rows = 64

    def step(c, carry):
        tb = wid * per + c * rows
        pltpu.sync_copy(xf_hbm.at[pl.ds(tb, rows)], rows_v)
        pltpu.sync_copy(d0_hbm.at[pl.ds(tb, rows)], idx_v)
        pltpu.async_copy(rows_v, xg_hbm.at[idx_v], sem).wait()
        pltpu.sync_copy(d1_hbm.at[pl.ds(tb, rows)], idx_v)
        pltpu.async_copy(rows_v, xg_hbm.at[idx_v], sem).wait()
        return carry

    lax.fori_loop(0, per // rows, step, 0)


@functools.cache
def _scatter_call():
    return pl.kernel(
        _scatter_body,
        out_type=jax.ShapeDtypeStruct((_PADT, _DIM), _F32),
        mesh=plsc.VectorSubcoreMesh(core_axis_name="c", subcore_axis_name="s"),
        scratch_types=[
            pltpu.VMEM((64,), jnp.int32),
            pltpu.VMEM((64, _DIM), _F32),
            pltpu.SemaphoreType.DMA,
        ],
    )


def _ffn_body(be_ref, xg_ref, w1_ref, w3_ref, w2_ref, y_ref):
    j = pl.program_id(0)

    @pl.when(j < be_ref[127])
    def _active():
        _ffn_compute(xg_ref, w1_ref, w3_ref, w2_ref, y_ref)


def _ffn_compute(xg_ref, w1_ref, w3_ref, w2_ref, y_ref):
    x = xg_ref[...]
    y_ref[...] = jnp.zeros_like(y_ref)
    for h in range(_H // _CH):
        w1c = w1_ref[0, pl.ds(h * _CH, _CH), :]
        w3c = w3_ref[0, pl.ds(h * _CH, _CH), :]
        a = lax.dot_general(x, w1c, (((1,), (1,)), ((), ())),
                            preferred_element_type=_F32)
        g = lax.dot_general(x, w3c, (((1,), (1,)), ((), ())),
                            preferred_element_type=_F32)
        hh = a * lax.logistic(a) * g
        w2c = w2_ref[0, :, pl.ds(h * _CH, _CH)]
        y_ref[...] += lax.dot_general(hh, w2c, (((1,), (1,)), ((), ())),
                                      preferred_element_type=_F32)


def _ffn_call(bexp, xg, w1, w3, w2):
    grid_spec = pltpu.PrefetchScalarGridSpec(
        num_scalar_prefetch=1,
        grid=(_NB,),
        in_specs=[
            pl.BlockSpec((_BLK, _DIM), lambda j, be: (j, 0)),
            pl.BlockSpec((1, _H, _DIM), lambda j, be: (be[j], 0, 0)),
            pl.BlockSpec((1, _H, _DIM), lambda j, be: (be[j], 0, 0)),
            pl.BlockSpec((1, _DIM, _H), lambda j, be: (be[j], 0, 0),
                         pipeline_mode=pl.Buffered(buffer_count=1)),
        ],
        out_specs=pl.BlockSpec((_BLK, _DIM), lambda j, be: (j, 0)),
    )
    return pl.pallas_call(
        _ffn_body,
        grid_spec=grid_spec,
        out_shape=jax.ShapeDtypeStruct((_PADT, _DIM), _F32),
    )(bexp, xg, w1, w3, w2)


def _combine_body(y_hbm, d0_hbm, d1_hbm, w0_hbm, w1_hbm, out_hbm,
                  i0_v, i1_v, w0_v, w1_v, r0_v, r1_v, o_v, sem):
    wid = lax.axis_index("s") * 2 + lax.axis_index("c")
    per = _N // _NW
    rows = 32
    dnums = lax.GatherDimensionNumbers(
        offset_dims=(), collapsed_slice_dims=(0,), start_index_map=(0,))

    def chunk(c, carry):
        tb = wid * per + c * rows
        pltpu.sync_copy(d0_hbm.at[pl.ds(tb, rows)], i0_v)
        pltpu.sync_copy(d1_hbm.at[pl.ds(tb, rows)], i1_v)
        pltpu.sync_copy(w0_hbm.at[pl.ds(tb, rows)], w0_v)
        pltpu.sync_copy(w1_hbm.at[pl.ds(tb, rows)], w1_v)
        pltpu.async_copy(y_hbm.at[i0_v], r0_v, sem).wait()
        pltpu.async_copy(y_hbm.at[i1_v], r1_v, sem).wait()

        def trow(t, tc):
            g = (t // 16) * 16
            wv0 = w0_v[pl.ds(g, 16)]
            wv1 = w1_v[pl.ds(g, 16)]
            tsplat = jnp.full((16, 1), t - g, jnp.int32)
            wt0 = lax.gather(wv0, tsplat, dnums, (1,),
                             mode=lax.GatherScatterMode.PROMISE_IN_BOUNDS)
            wt1 = lax.gather(wv1, tsplat, dnums, (1,),
                             mode=lax.GatherScatterMode.PROMISE_IN_BOUNDS)

            def col(jj, jc):
                sl = pl.ds(jj * 16, 16)
                o_v[t, sl] = wt0 * r0_v[t, sl] + wt1 * r1_v[t, sl]
                return jc

            lax.fori_loop(0, _DIM // 16, col, 0, unroll=8)
            return tc

        lax.fori_loop(0, rows, trow, 0)
        pltpu.sync_copy(o_v, out_hbm.at[pl.ds(tb, rows)])
        return carry

    lax.fori_loop(0, per // rows, chunk, 0)


@functools.cache
def _combine_call():
    return pl.kernel(
        _combine_body,
        out_type=jax.ShapeDtypeStruct((_N, _DIM), _F32),
        mesh=plsc.VectorSubcoreMesh(core_axis_name="c", subcore_axis_name="s"),
        scratch_types=[
            pltpu.VMEM((32,), jnp.int32),
            pltpu.VMEM((32,), jnp.int32),
            pltpu.VMEM((32,), _F32),
            pltpu.VMEM((32,), _F32),
            pltpu.VMEM((32, _DIM), _F32),
            pltpu.VMEM((32, _DIM), _F32),
            pltpu.VMEM((32, _DIM), _F32),
            pltpu.SemaphoreType.DMA,
        ],
    )


def kernel(x, gate_W, w1, w2, w3):
    orig_shape = x.shape
    xf = x.reshape(_N, _DIM)
    d0, d1, w0p, w1p, be, aux = _gate_call(xf, gate_W)
    d0f = d0.reshape(_N)
    d1f = d1.reshape(_N)
    w0f = w0p.reshape(_N)
    w1f = w1p.reshape(_N)
    bexp = be.reshape(128)
    xg = _scatter_call()(xf, d0f, d1f)
    y = _ffn_call(bexp, xg, w1, w3, w2)
    out = _combine_call()(y, d0f, d1f, w0f, w1f)
    return out.reshape(orig_shape), aux.reshape(())


# speedup vs baseline: 1.1080x; 1.1080x over previous
import functools

import jax
import jax.numpy as jnp
from jax import lax
from jax.experimental import pallas as pl
from jax.experimental.pallas import tpu as pltpu
from jax.experimental.pallas import tpu_sc as plsc

_DIM = 1024
_E = 8
_H = 2560
_N = 8192
_T = 256
_NBT = _N // _T
_BLK = 256
_NB = (2 * _N) // _BLK + _E
_PADT = _NB * _BLK
_CH = 512
_NW = 32
_F32 = jnp.float32


def _gate_body(x_ref, gw_ref, d0_ref, d1_ref, w0_ref, w1_ref, be_ref,
               aux_ref, acc_ref, imp_ref, ident_ref, ltri_ref):
    ph = pl.program_id(0)
    b = pl.program_id(1)

    @pl.when((ph == 0) & (b == 0))
    def _init():
        acc_ref[...] = jnp.zeros_like(acc_ref)
        imp_ref[...] = jnp.zeros_like(imp_ref)
        ident_ref[...] = (
            lax.broadcasted_iota(jnp.int32, (_T, _T), 0)
            == lax.broadcasted_iota(jnp.int32, (_T, _T), 1)).astype(_F32)
        ltri_ref[...] = (
            lax.broadcasted_iota(jnp.int32, (2 * _T, 2 * _T), 0)
            < lax.broadcasted_iota(jnp.int32, (2 * _T, 2 * _T), 1)).astype(_F32)

    x = x_ref[...]
    gw = gw_ref[...]
    logits = lax.dot_general(x, gw, (((1,), (1,)), ((), ())),
                             preferred_element_type=_F32)
    m = jnp.max(logits, axis=1, keepdims=True)
    ex = jnp.exp(logits - m)
    probs = ex / jnp.sum(ex, axis=1, keepdims=True)

    ei = lax.broadcasted_iota(jnp.int32, (_T, _E), 1).astype(_F32)
    m1 = jnp.max(probs, axis=1, keepdims=True)
    i1 = jnp.min(jnp.where(probs == m1, ei, 8.0), axis=1, keepdims=True)
    oh1 = (ei == i1).astype(_F32)
    pm = probs - 2.0 * oh1
    m2 = jnp.max(pm, axis=1, keepdims=True)
    i2 = jnp.min(jnp.where(pm == m2, ei, 8.0), axis=1, keepdims=True)

    icat = jnp.concatenate([i1, i2], axis=1)
    it2 = lax.dot_general(icat, ident_ref[...], (((0,), (0,)), ((), ())),
                          preferred_element_type=_F32)
    e8 = lax.broadcasted_iota(jnp.int32, (_E, 2 * _T), 0).astype(_F32)
    ipairs = jnp.concatenate([it2[0:1, :], it2[1:2, :]], axis=1)
    oht = (e8 == ipairs).astype(_F32)

    @pl.when(ph == 0)
    def _count():
        acc_ref[:, 0:1] += jnp.sum(oht, axis=1, keepdims=True)
        imp_ref[0:1, 0:8] += jnp.sum(probs, axis=0, keepdims=True)

    @pl.when((ph == 0) & (b == _NBT - 1))
    def _offsets():
        cnt = acc_ref[:, 0:1]
        nb = jnp.floor((cnt + (_BLK - 1.0)) * (1.0 / _BLK))
        tril8 = (lax.broadcasted_iota(jnp.int32, (_E, _E), 0)
                 > lax.broadcasted_iota(jnp.int32, (_E, _E), 1)).astype(_F32)
        offs = lax.dot_general(tril8, nb, (((1,), (0,)), ((), ())),
                               preferred_element_type=_F32)
        acc_ref[:, 2:3] = offs * float(_BLK)
        acc_ref[:, 1:2] = jnp.zeros((_E, 1), _F32)
        acc_ref[:, 3:4] = offs + nb

    @pl.when(ph == 1)
    def _rank():
        ranks = lax.dot_general(oht, ltri_ref[...], (((1,), (0,)), ((), ())),
                                preferred_element_type=_F32)
        rank = jnp.sum(oht * ranks, axis=0, keepdims=True)
        basecol = acc_ref[:, 2:3] + acc_ref[:, 1:2]
        base = jnp.sum(oht * basecol, axis=0, keepdims=True)
        dest = base + rank
        d0_ref[...] = dest[:, 0:_T].astype(jnp.int32).reshape(1, 1, _T)
        d1_ref[...] = dest[:, _T:2 * _T].astype(jnp.int32).reshape(1, 1, _T)
        w0_ref[...] = m1 / (m1 + m2)
        w1_ref[...] = m2 / (m1 + m2)
        acc_ref[:, 1:2] += jnp.sum(oht, axis=1, keepdims=True)

    @pl.when((ph == 1) & (b == _NBT - 1))
    def _finish():
        aux = lax.dot_general(imp_ref[0:1, 0:8], acc_ref[:, 0:1],
                              (((1,), (0,)), ((), ())),
                              preferred_element_type=_F32,
                              precision=lax.Precision.HIGHEST)
        aux_ref[...] = aux * (float(_E) / (float(_N) * float(_N)))
        jv = lax.broadcasted_iota(jnp.int32, (1, 128), 1).astype(_F32)
        bx = jnp.sum((acc_ref[:, 3:4] <= jv).astype(_F32), axis=0,
                     keepdims=True)
        used = jnp.sum(acc_ref[7:8, 3:4])
        bxx = jnp.where(jv >= 127.0, used, jnp.minimum(bx, 7.0))
        be_ref[...] = bxx.astype(jnp.int32)


def _gate_call(xf, gate_w):
    return pl.pallas_call(
        _gate_body,
        grid=(2, _NBT),
        in_specs=[
            pl.BlockSpec((_T, _DIM), lambda p, b: (b, 0)),
            pl.BlockSpec((_E, _DIM), lambda p, b: (0, 0)),
        ],
        out_specs=[
            pl.BlockSpec((1, 1, _T), lambda p, b: (b, 0, 0)),
            pl.BlockSpec((1, 1, _T), lambda p, b: (b, 0, 0)),
            pl.BlockSpec((_T, 1), lambda p, b: (b, 0)),
            pl.BlockSpec((_T, 1), lambda p, b: (b, 0)),
            pl.BlockSpec((1, 128), lambda p, b: (0, 0)),
            pl.BlockSpec((1, 1), lambda p, b: (0, 0)),
        ],
        out_shape=[
            jax.ShapeDtypeStruct((_NBT, 1, _T), jnp.int32),
            jax.ShapeDtypeStruct((_NBT, 1, _T), jnp.int32),
            jax.ShapeDtypeStruct((_N, 1), _F32),
            jax.ShapeDtypeStruct((_N, 1), _F32),
            jax.ShapeDtypeStruct((1, 128), jnp.int32),
            jax.ShapeDtypeStruct((1, 1), _F32),
        ],
        scratch_shapes=[pltpu.VMEM((_E, 128), _F32),
                        pltpu.VMEM((_E, 128), _F32),
                        pltpu.VMEM((_T, _T), _F32),
                        pltpu.VMEM((2 * _T, 2 * _T), _F32)],
    )(xf, gate_w)


def _scatter_body(xf_hbm, d0_hbm, d1_hbm, xg_hbm, idx_v, rows_v, sem):
    wid = lax.axis_index("s") * 2 + lax.axis_index("c")
    per = _N // _NW
    rows = 64

    def step(c, carry):
        tb = wid * per + c * rows
        pltpu.sync_copy(xf_hbm.at[pl.ds(tb, rows)], rows_v)
        pltpu.sync_copy(d0_hbm.at[pl.ds(tb, rows)], idx_v)
        pltpu.async_copy(rows_v, xg_hbm.at[idx_v], sem).wait()
        pltpu.sync_copy(d1_hbm.at[pl.ds(tb, rows)], idx_v)
        pltpu.async_copy(rows_v, xg_hbm.at[idx_v], sem).wait()
        return carry

    lax.fori_loop(0, per // rows, step, 0)


@functools.cache
def _scatter_call():
    return pl.kernel(
        _scatter_body,
        out_type=jax.ShapeDtypeStruct((_PADT, _DIM), _F32),
        mesh=plsc.VectorSubcoreMesh(core_axis_name="c", subcore_axis_name="s"),
        scratch_types=[
            pltpu.VMEM((64,), jnp.int32),
            pltpu.VMEM((64, _DIM), _F32),
            pltpu.SemaphoreType.DMA,
        ],
    )


def _ffn_body(be_ref, xg_ref, w1_ref, w3_ref, w2_ref, y_ref):
    j = pl.program_id(0)

    @pl.when(j < be_ref[127])
    def _active():
        _ffn_compute(xg_ref, w1_ref, w3_ref, w2_ref, y_ref)


def _ffn_compute(xg_ref, w1_ref, w3_ref, w2_ref, y_ref):
    x = xg_ref[...]
    y_ref[...] = jnp.zeros_like(y_ref)
    for h in range(_H // _CH):
        w1c = w1_ref[0, pl.ds(h * _CH, _CH), :]
        w3c = w3_ref[0, pl.ds(h * _CH, _CH), :]
        a = lax.dot_general(x, w1c, (((1,), (1,)), ((), ())),
                            preferred_element_type=_F32)
        g = lax.dot_general(x, w3c, (((1,), (1,)), ((), ())),
                            preferred_element_type=_F32)
        hh = a * lax.logistic(a) * g
        w2c = w2_ref[0, :, pl.ds(h * _CH, _CH)]
        y_ref[...] += lax.dot_general(hh, w2c, (((1,), (1,)), ((), ())),
                                      preferred_element_type=_F32)


def _ffn_call(bexp, xg, w1, w3, w2):
    grid_spec = pltpu.PrefetchScalarGridSpec(
        num_scalar_prefetch=1,
        grid=(_NB,),
        in_specs=[
            pl.BlockSpec((_BLK, _DIM), lambda j, be: (j, 0)),
            pl.BlockSpec((1, _H, _DIM), lambda j, be: (be[j], 0, 0)),
            pl.BlockSpec((1, _H, _DIM), lambda j, be: (be[j], 0, 0)),
            pl.BlockSpec((1, _DIM, _H), lambda j, be: (be[j], 0, 0),
                         pipeline_mode=pl.Buffered(buffer_count=1)),
        ],
        out_specs=pl.BlockSpec((_BLK, _DIM), lambda j, be: (j, 0)),
    )
    return pl.pallas_call(
        _ffn_body,
        grid_spec=grid_spec,
        out_shape=jax.ShapeDtypeStruct((_PADT, _DIM), _F32),
    )(bexp, xg, w1, w3, w2)


def _combine_body(y_hbm, d0_hbm, d1_hbm, o0_hbm, o1_hbm,
                  i0_v, i1_v, r0_v, r1_v, sem, sem2):
    wid = lax.axis_index("s") * 2 + lax.axis_index("c")
    per = _N // _NW
    rows = 32

    def chunk(c, carry):
        tb = wid * per + c * rows
        pltpu.sync_copy(d0_hbm.at[pl.ds(tb, rows)], i0_v)
        pltpu.sync_copy(d1_hbm.at[pl.ds(tb, rows)], i1_v)
        cp0 = pltpu.async_copy(y_hbm.at[i0_v], r0_v, sem)
        cp1 = pltpu.async_copy(y_hbm.at[i1_v], r1_v, sem2)
        cp0.wait()
        pltpu.sync_copy(r0_v, o0_hbm.at[pl.ds(tb, rows)])
        cp1.wait()
        pltpu.sync_copy(r1_v, o1_hbm.at[pl.ds(tb, rows)])
        return carry

    lax.fori_loop(0, per // rows, chunk, 0)


@functools.cache
def _combine_call():
    return pl.kernel(
        _combine_body,
        out_type=(jax.ShapeDtypeStruct((_N, _DIM), _F32),
                  jax.ShapeDtypeStruct((_N, _DIM), _F32)),
        mesh=plsc.VectorSubcoreMesh(core_axis_name="c", subcore_axis_name="s"),
        scratch_types=[
            pltpu.VMEM((32,), jnp.int32),
            pltpu.VMEM((32,), jnp.int32),
            pltpu.VMEM((32, _DIM), _F32),
            pltpu.VMEM((32, _DIM), _F32),
            pltpu.SemaphoreType.DMA,
            pltpu.SemaphoreType.DMA,
        ],
    )


def _mix_body(w0_ref, w1_ref, a_ref, b_ref, o_ref):
    o_ref[...] = w0_ref[...] * a_ref[...] + w1_ref[...] * b_ref[...]


def _mix_call(w0c, w1c, o0, o1):
    tm = 1024
    return pl.pallas_call(
        _mix_body,
        grid=(_N // tm,),
        in_specs=[
            pl.BlockSpec((tm, 1), lambda i: (i, 0)),
            pl.BlockSpec((tm, 1), lambda i: (i, 0)),
            pl.BlockSpec((tm, _DIM), lambda i: (i, 0)),
            pl.BlockSpec((tm, _DIM), lambda i: (i, 0)),
        ],
        out_specs=pl.BlockSpec((tm, _DIM), lambda i: (i, 0)),
        out_shape=jax.ShapeDtypeStruct((_N, _DIM), _F32),
    )(w0c, w1c, o0, o1)


def kernel(x, gate_W, w1, w2, w3):
    orig_shape = x.shape
    xf = x.reshape(_N, _DIM)
    d0, d1, w0c, w1c, be, aux = _gate_call(xf, gate_W)
    d0f = d0.reshape(_N)
    d1f = d1.reshape(_N)
    bexp = be.reshape(128)
    xg = _scatter_call()(xf, d0f, d1f)
    y = _ffn_call(bexp, xg, w1, w3, w2)
    o0, o1 = _combine_call()(y, d0f, d1f)
    out = _mix_call(w0c, w1c, o0, o1)
    return out.reshape(orig_shape), aux.reshape(())


# speedup vs baseline: 1.1947x; 1.0782x over previous
import functools

import jax
import jax.numpy as jnp
from jax import lax
from jax.experimental import pallas as pl
from jax.experimental.pallas import tpu as pltpu
from jax.experimental.pallas import tpu_sc as plsc

_DIM = 1024
_E = 8
_H = 2560
_N = 8192
_T = 256
_NBT = _N // _T
_BLK = 256
_NB = (2 * _N) // _BLK + _E
_PADT = _NB * _BLK
_CH = 512
_NW = 32
_F32 = jnp.float32


def _gate_body(x_ref, gw_ref, d0_ref, d1_ref, w0_ref, w1_ref, be_ref,
               aux_ref, acc_ref, imp_ref, ident_ref, ltri_ref, sm_ref):
    ph = pl.program_id(0)
    b = pl.program_id(1)

    @pl.when((ph == 0) & (b == 0))
    def _init():
        acc_ref[...] = jnp.zeros_like(acc_ref)
        imp_ref[...] = jnp.zeros_like(imp_ref)
        ident_ref[...] = (
            lax.broadcasted_iota(jnp.int32, (_T, _T), 0)
            == lax.broadcasted_iota(jnp.int32, (_T, _T), 1)).astype(_F32)
        ltri_ref[...] = (
            lax.broadcasted_iota(jnp.int32, (2 * _T, 2 * _T), 0)
            < lax.broadcasted_iota(jnp.int32, (2 * _T, 2 * _T), 1)).astype(_F32)

    def _make_oht(icat):
        it2 = lax.dot_general(icat, ident_ref[...], (((0,), (0,)), ((), ())),
                              preferred_element_type=_F32)
        e8 = lax.broadcasted_iota(jnp.int32, (_E, 2 * _T), 0).astype(_F32)
        ipairs = jnp.concatenate([it2[0:1, :], it2[1:2, :]], axis=1)
        return (e8 == ipairs).astype(_F32)

    @pl.when(ph == 0)
    def _count():
        x = x_ref[...]
        gw = gw_ref[...]
        logits = lax.dot_general(x, gw, (((1,), (1,)), ((), ())),
                                 preferred_element_type=_F32)
        m = jnp.max(logits, axis=1, keepdims=True)
        ex = jnp.exp(logits - m)
        probs = ex / jnp.sum(ex, axis=1, keepdims=True)

        ei = lax.broadcasted_iota(jnp.int32, (_T, _E), 1).astype(_F32)
        m1 = jnp.max(probs, axis=1, keepdims=True)
        i1 = jnp.min(jnp.where(probs == m1, ei, 8.0), axis=1, keepdims=True)
        oh1 = (ei == i1).astype(_F32)
        pm = probs - 2.0 * oh1
        m2 = jnp.max(pm, axis=1, keepdims=True)
        i2 = jnp.min(jnp.where(pm == m2, ei, 8.0), axis=1, keepdims=True)
        sm_ref[pl.ds(b * _T, _T), :] = jnp.concatenate(
            [i1, i2, m1, m2], axis=1)
        oht = _make_oht(jnp.concatenate([i1, i2], axis=1))
        acc_ref[:, 0:1] += jnp.sum(oht, axis=1, keepdims=True)
        imp_ref[0:1, 0:8] += jnp.sum(probs, axis=0, keepdims=True)

    @pl.when((ph == 0) & (b == _NBT - 1))
    def _offsets():
        cnt = acc_ref[:, 0:1]
        nb = jnp.floor((cnt + (_BLK - 1.0)) * (1.0 / _BLK))
        tril8 = (lax.broadcasted_iota(jnp.int32, (_E, _E), 0)
                 > lax.broadcasted_iota(jnp.int32, (_E, _E), 1)).astype(_F32)
        offs = lax.dot_general(tril8, nb, (((1,), (0,)), ((), ())),
                               preferred_element_type=_F32)
        acc_ref[:, 2:3] = offs * float(_BLK)
        acc_ref[:, 1:2] = jnp.zeros((_E, 1), _F32)
        acc_ref[:, 3:4] = offs + nb

    @pl.when(ph == 1)
    def _rank():
        sm = sm_ref[pl.ds(b * _T, _T), :]
        m1 = sm[:, 2:3]
        m2 = sm[:, 3:4]
        oht = _make_oht(sm[:, 0:2])
        ranks = lax.dot_general(oht, ltri_ref[...], (((1,), (0,)), ((), ())),
                                preferred_element_type=_F32)
        rank = jnp.sum(oht * ranks, axis=0, keepdims=True)
        basecol = acc_ref[:, 2:3] + acc_ref[:, 1:2]
        base = jnp.sum(oht * basecol, axis=0, keepdims=True)
        dest = base + rank
        d0_ref[...] = dest[:, 0:_T].astype(jnp.int32).reshape(1, 1, _T)
        d1_ref[...] = dest[:, _T:2 * _T].astype(jnp.int32).reshape(1, 1, _T)
        w0_ref[...] = m1 / (m1 + m2)
        w1_ref[...] = m2 / (m1 + m2)
        acc_ref[:, 1:2] += jnp.sum(oht, axis=1, keepdims=True)

    @pl.when((ph == 1) & (b == _NBT - 1))
    def _finish():
        aux = lax.dot_general(imp_ref[0:1, 0:8], acc_ref[:, 0:1],
                              (((1,), (0,)), ((), ())),
                              preferred_element_type=_F32,
                              precision=lax.Precision.HIGHEST)
        aux_ref[...] = aux * (float(_E) / (float(_N) * float(_N)))
        jv = lax.broadcasted_iota(jnp.int32, (1, 128), 1).astype(_F32)
        bx = jnp.sum((acc_ref[:, 3:4] <= jv).astype(_F32), axis=0,
                     keepdims=True)
        used = jnp.sum(acc_ref[7:8, 3:4])
        bxx = jnp.where(jv >= 127.0, used, jnp.minimum(bx, 7.0))
        be_ref[...] = bxx.astype(jnp.int32)


def _gate_call(xf, gate_w):
    return pl.pallas_call(
        _gate_body,
        grid=(2, _NBT),
        in_specs=[
            pl.BlockSpec((_T, _DIM), lambda p, b: (b * (1 - p), 0)),
            pl.BlockSpec((_E, _DIM), lambda p, b: (0, 0)),
        ],
        out_specs=[
            pl.BlockSpec((1, 1, _T), lambda p, b: (b, 0, 0)),
            pl.BlockSpec((1, 1, _T), lambda p, b: (b, 0, 0)),
            pl.BlockSpec((_T, 1), lambda p, b: (b, 0)),
            pl.BlockSpec((_T, 1), lambda p, b: (b, 0)),
            pl.BlockSpec((1, 128), lambda p, b: (0, 0)),
            pl.BlockSpec((1, 1), lambda p, b: (0, 0)),
        ],
        out_shape=[
            jax.ShapeDtypeStruct((_NBT, 1, _T), jnp.int32),
            jax.ShapeDtypeStruct((_NBT, 1, _T), jnp.int32),
            jax.ShapeDtypeStruct((_N, 1), _F32),
            jax.ShapeDtypeStruct((_N, 1), _F32),
            jax.ShapeDtypeStruct((1, 128), jnp.int32),
            jax.ShapeDtypeStruct((1, 1), _F32),
        ],
        scratch_shapes=[pltpu.VMEM((_E, 128), _F32),
                        pltpu.VMEM((_E, 128), _F32),
                        pltpu.VMEM((_T, _T), _F32),
                        pltpu.VMEM((2 * _T, 2 * _T), _F32),
                        pltpu.VMEM((_N, 4), _F32)],
    )(xf, gate_w)


def _scatter_body(xf_hbm, d0_hbm, d1_hbm, xg_hbm, idx_v, rows_v, sem):
    wid = lax.axis_index("s") * 2 + lax.axis_index("c")
    per = _N // _NW
    rows = 64

    def step(c, carry):
        tb = wid * per + c * rows
        pltpu.sync_copy(xf_hbm.at[pl.ds(tb, rows)], rows_v)
        pltpu.sync_copy(d0_hbm.at[pl.ds(tb, rows)], idx_v)
        pltpu.async_copy(rows_v, xg_hbm.at[idx_v], sem).wait()
        pltpu.sync_copy(d1_hbm.at[pl.ds(tb, rows)], idx_v)
        pltpu.async_copy(rows_v, xg_hbm.at[idx_v], sem).wait()
        return carry

    lax.fori_loop(0, per // rows, step, 0)


@functools.cache
def _scatter_call():
    return pl.kernel(
        _scatter_body,
        out_type=jax.ShapeDtypeStruct((_PADT, _DIM), _F32),
        mesh=plsc.VectorSubcoreMesh(core_axis_name="c", subcore_axis_name="s"),
        scratch_types=[
            pltpu.VMEM((64,), jnp.int32),
            pltpu.VMEM((64, _DIM), _F32),
            pltpu.SemaphoreType.DMA,
        ],
    )


def _ffn_body(be_ref, xg_ref, w1_ref, w3_ref, w2_ref, y_ref, h_ref):
    j = pl.program_id(0)

    @pl.when(j < be_ref[127])
    def _active():
        _ffn_compute(xg_ref, w1_ref, w3_ref, w2_ref, y_ref, h_ref)


def _ffn_compute(xg_ref, w1_ref, w3_ref, w2_ref, y_ref, h_ref):
    x = xg_ref[...]
    for h in range(_H // _CH):
        w1c = w1_ref[0, pl.ds(h * _CH, _CH), :]
        w3c = w3_ref[0, pl.ds(h * _CH, _CH), :]
        a = lax.dot_general(x, w1c, (((1,), (1,)), ((), ())),
                            preferred_element_type=_F32)
        g = lax.dot_general(x, w3c, (((1,), (1,)), ((), ())),
                            preferred_element_type=_F32)
        h_ref[:, pl.ds(h * _CH, _CH)] = a * lax.logistic(a) * g
    y_ref[...] = lax.dot_general(h_ref[...], w2_ref[0], (((1,), (1,)), ((), ())),
                                 preferred_element_type=_F32)


def _ffn_call(bexp, xg, w1, w3, w2):
    grid_spec = pltpu.PrefetchScalarGridSpec(
        num_scalar_prefetch=1,
        grid=(_NB,),
        in_specs=[
            pl.BlockSpec((_BLK, _DIM), lambda j, be: (j, 0)),
            pl.BlockSpec((1, _H, _DIM), lambda j, be: (be[j], 0, 0)),
            pl.BlockSpec((1, _H, _DIM), lambda j, be: (be[j], 0, 0)),
            pl.BlockSpec((1, _DIM, _H), lambda j, be: (be[j], 0, 0),
                         pipeline_mode=pl.Buffered(buffer_count=1)),
        ],
        out_specs=pl.BlockSpec((_BLK, _DIM), lambda j, be: (j, 0)),
        scratch_shapes=[pltpu.VMEM((_BLK, _H), _F32)],
    )
    return pl.pallas_call(
        _ffn_body,
        grid_spec=grid_spec,
        out_shape=jax.ShapeDtypeStruct((_PADT, _DIM), _F32),
    )(bexp, xg, w1, w3, w2)


def _combine_body(y_hbm, d0_hbm, d1_hbm, o0_hbm, o1_hbm,
                  i0_v, i1_v, r0_v, r1_v, sem, sem2):
    wid = lax.axis_index("s") * 2 + lax.axis_index("c")
    per = _N // _NW
    rows = 32

    def chunk(c, carry):
        tb = wid * per + c * rows
        pltpu.sync_copy(d0_hbm.at[pl.ds(tb, rows)], i0_v)
        pltpu.sync_copy(d1_hbm.at[pl.ds(tb, rows)], i1_v)
        cp0 = pltpu.async_copy(y_hbm.at[i0_v], r0_v, sem)
        cp1 = pltpu.async_copy(y_hbm.at[i1_v], r1_v, sem2)
        cp0.wait()
        pltpu.sync_copy(r0_v, o0_hbm.at[pl.ds(tb, rows)])
        cp1.wait()
        pltpu.sync_copy(r1_v, o1_hbm.at[pl.ds(tb, rows)])
        return carry

    lax.fori_loop(0, per // rows, chunk, 0)


@functools.cache
def _combine_call():
    return pl.kernel(
        _combine_body,
        out_type=(jax.ShapeDtypeStruct((_N, _DIM), _F32),
                  jax.ShapeDtypeStruct((_N, _DIM), _F32)),
        mesh=plsc.VectorSubcoreMesh(core_axis_name="c", subcore_axis_name="s"),
        scratch_types=[
            pltpu.VMEM((32,), jnp.int32),
            pltpu.VMEM((32,), jnp.int32),
            pltpu.VMEM((32, _DIM), _F32),
            pltpu.VMEM((32, _DIM), _F32),
            pltpu.SemaphoreType.DMA,
            pltpu.SemaphoreType.DMA,
        ],
    )


def _mix_body(w0_ref, w1_ref, a_ref, b_ref, o_ref):
    o_ref[...] = w0_ref[...] * a_ref[...] + w1_ref[...] * b_ref[...]


def _mix_call(w0c, w1c, o0, o1):
    tm = 1024
    return pl.pallas_call(
        _mix_body,
        grid=(_N // tm,),
        in_specs=[
            pl.BlockSpec((tm, 1), lambda i: (i, 0)),
            pl.BlockSpec((tm, 1), lambda i: (i, 0)),
            pl.BlockSpec((tm, _DIM), lambda i: (i, 0)),
            pl.BlockSpec((tm, _DIM), lambda i: (i, 0)),
        ],
        out_specs=pl.BlockSpec((tm, _DIM), lambda i: (i, 0)),
        out_shape=jax.ShapeDtypeStruct((_N, _DIM), _F32),
    )(w0c, w1c, o0, o1)


def kernel(x, gate_W, w1, w2, w3):
    orig_shape = x.shape
    xf = x.reshape(_N, _DIM)
    d0, d1, w0c, w1c, be, aux = _gate_call(xf, gate_W)
    d0f = d0.reshape(_N)
    d1f = d1.reshape(_N)
    bexp = be.reshape(128)
    xg = _scatter_call()(xf, d0f, d1f)
    y = _ffn_call(bexp, xg, w1, w3, w2)
    o0, o1 = _combine_call()(y, d0f, d1f)
    out = _mix_call(w0c, w1c, o0, o1)
    return out.reshape(orig_shape), aux.reshape(())


# speedup vs baseline: 1.2345x; 1.0333x over previous
import functools

import jax
import jax.numpy as jnp
from jax import lax
from jax.experimental import pallas as pl
from jax.experimental.pallas import tpu as pltpu
from jax.experimental.pallas import tpu_sc as plsc

_DIM = 1024
_E = 8
_H = 2560
_N = 8192
_T = 512
_NBT = _N // _T
_BLK = 256
_NB = (2 * _N) // _BLK + _E
_PADT = _NB * _BLK
_CH = 512
_NW = 32
_F32 = jnp.float32


def _gate_body(x_ref, gw_ref, d0_ref, d1_ref, w0_ref, w1_ref, be_ref,
               aux_ref, acc_ref, imp_ref, ident_ref, ltri_ref, sm_ref):
    ph = pl.program_id(0)
    b = pl.program_id(1)

    @pl.when((ph == 0) & (b == 0))
    def _init():
        acc_ref[...] = jnp.zeros_like(acc_ref)
        imp_ref[...] = jnp.zeros_like(imp_ref)
        ident_ref[...] = (
            lax.broadcasted_iota(jnp.int32, (_T, _T), 0)
            == lax.broadcasted_iota(jnp.int32, (_T, _T), 1)).astype(_F32)
        ltri_ref[...] = (
            lax.broadcasted_iota(jnp.int32, (2 * _T, 2 * _T), 0)
            < lax.broadcasted_iota(jnp.int32, (2 * _T, 2 * _T), 1)).astype(_F32)

    def _make_oht(icat):
        it2 = lax.dot_general(icat, ident_ref[...], (((0,), (0,)), ((), ())),
                              preferred_element_type=_F32)
        e8 = lax.broadcasted_iota(jnp.int32, (_E, 2 * _T), 0).astype(_F32)
        ipairs = jnp.concatenate([it2[0:1, :], it2[1:2, :]], axis=1)
        return (e8 == ipairs).astype(_F32)

    @pl.when(ph == 0)
    def _count():
        x = x_ref[...]
        gw = gw_ref[...]
        logits = lax.dot_general(x, gw, (((1,), (1,)), ((), ())),
                                 preferred_element_type=_F32)
        m = jnp.max(logits, axis=1, keepdims=True)
        ex = jnp.exp(logits - m)
        probs = ex / jnp.sum(ex, axis=1, keepdims=True)

        ei = lax.broadcasted_iota(jnp.int32, (_T, _E), 1).astype(_F32)
        m1 = jnp.max(probs, axis=1, keepdims=True)
        i1 = jnp.min(jnp.where(probs == m1, ei, 8.0), axis=1, keepdims=True)
        oh1 = (ei == i1).astype(_F32)
        pm = probs - 2.0 * oh1
        m2 = jnp.max(pm, axis=1, keepdims=True)
        i2 = jnp.min(jnp.where(pm == m2, ei, 8.0), axis=1, keepdims=True)
        sm_ref[pl.ds(b * _T, _T), :] = jnp.concatenate(
            [i1, i2, m1, m2], axis=1)
        oht = _make_oht(jnp.concatenate([i1, i2], axis=1))
        acc_ref[:, 0:1] += jnp.sum(oht, axis=1, keepdims=True)
        imp_ref[0:1, 0:8] += jnp.sum(probs, axis=0, keepdims=True)

    @pl.when((ph == 0) & (b == _NBT - 1))
    def _offsets():
        cnt = acc_ref[:, 0:1]
        nb = jnp.floor((cnt + (_BLK - 1.0)) * (1.0 / _BLK))
        tril8 = (lax.broadcasted_iota(jnp.int32, (_E, _E), 0)
                 > lax.broadcasted_iota(jnp.int32, (_E, _E), 1)).astype(_F32)
        offs = lax.dot_general(tril8, nb, (((1,), (0,)), ((), ())),
                               preferred_element_type=_F32)
        acc_ref[:, 2:3] = offs * float(_BLK)
        acc_ref[:, 1:2] = jnp.zeros((_E, 1), _F32)
        acc_ref[:, 3:4] = offs + nb

    @pl.when(ph == 1)
    def _rank():
        sm = sm_ref[pl.ds(b * _T, _T), :]
        m1 = sm[:, 2:3]
        m2 = sm[:, 3:4]
        oht = _make_oht(sm[:, 0:2])
        ranks = lax.dot_general(oht, ltri_ref[...], (((1,), (0,)), ((), ())),
                                preferred_element_type=_F32)
        rank = jnp.sum(oht * ranks, axis=0, keepdims=True)
        basecol = acc_ref[:, 2:3] + acc_ref[:, 1:2]
        base = jnp.sum(oht * basecol, axis=0, keepdims=True)
        dest = base + rank
        d0_ref[...] = dest[:, 0:_T].astype(jnp.int32).reshape(1, 1, _T)
        d1_ref[...] = dest[:, _T:2 * _T].astype(jnp.int32).reshape(1, 1, _T)
        w0_ref[...] = m1 / (m1 + m2)
        w1_ref[...] = m2 / (m1 + m2)
        acc_ref[:, 1:2] += jnp.sum(oht, axis=1, keepdims=True)

    @pl.when((ph == 1) & (b == _NBT - 1))
    def _finish():
        aux = lax.dot_general(imp_ref[0:1, 0:8], acc_ref[:, 0:1],
                              (((1,), (0,)), ((), ())),
                              preferred_element_type=_F32,
                              precision=lax.Precision.HIGHEST)
        aux_ref[...] = aux * (float(_E) / (float(_N) * float(_N)))
        jv = lax.broadcasted_iota(jnp.int32, (1, 128), 1).astype(_F32)
        bx = jnp.sum((acc_ref[:, 3:4] <= jv).astype(_F32), axis=0,
                     keepdims=True)
        used = jnp.sum(acc_ref[7:8, 3:4])
        bxx = jnp.where(jv >= 127.0, used, jnp.minimum(bx, 7.0))
        be_ref[...] = bxx.astype(jnp.int32)


def _gate_call(xf, gate_w):
    return pl.pallas_call(
        _gate_body,
        grid=(2, _NBT),
        in_specs=[
            pl.BlockSpec((_T, _DIM), lambda p, b: (b * (1 - p), 0)),
            pl.BlockSpec((_E, _DIM), lambda p, b: (0, 0)),
        ],
        out_specs=[
            pl.BlockSpec((1, 1, _T), lambda p, b: (b, 0, 0)),
            pl.BlockSpec((1, 1, _T), lambda p, b: (b, 0, 0)),
            pl.BlockSpec((_T, 1), lambda p, b: (b, 0)),
            pl.BlockSpec((_T, 1), lambda p, b: (b, 0)),
            pl.BlockSpec((1, 128), lambda p, b: (0, 0)),
            pl.BlockSpec((1, 1), lambda p, b: (0, 0)),
        ],
        out_shape=[
            jax.ShapeDtypeStruct((_NBT, 1, _T), jnp.int32),
            jax.ShapeDtypeStruct((_NBT, 1, _T), jnp.int32),
            jax.ShapeDtypeStruct((_N, 1), _F32),
            jax.ShapeDtypeStruct((_N, 1), _F32),
            jax.ShapeDtypeStruct((1, 128), jnp.int32),
            jax.ShapeDtypeStruct((1, 1), _F32),
        ],
        scratch_shapes=[pltpu.VMEM((_E, 128), _F32),
                        pltpu.VMEM((_E, 128), _F32),
                        pltpu.VMEM((_T, _T), _F32),
                        pltpu.VMEM((2 * _T, 2 * _T), _F32),
                        pltpu.VMEM((_N, 4), _F32)],
    )(xf, gate_w)


def _scatter_body(xf_hbm, d0_hbm, d1_hbm, xg_hbm, idx_v, rows_v, sem):
    wid = lax.axis_index("s") * 2 + lax.axis_index("c")
    per = _N // _NW
    rows = 64

    def step(c, carry):
        tb = wid * per + c * rows
        pltpu.sync_copy(xf_hbm.at[pl.ds(tb, rows)], rows_v)
        pltpu.sync_copy(d0_hbm.at[pl.ds(tb, rows)], idx_v)
        pltpu.async_copy(rows_v, xg_hbm.at[idx_v], sem).wait()
        pltpu.sync_copy(d1_hbm.at[pl.ds(tb, rows)], idx_v)
        pltpu.async_copy(rows_v, xg_hbm.at[idx_v], sem).wait()
        return carry

    lax.fori_loop(0, per // rows, step, 0)


@functools.cache
def _scatter_call():
    return pl.kernel(
        _scatter_body,
        out_type=jax.ShapeDtypeStruct((_PADT, _DIM), _F32),
        mesh=plsc.VectorSubcoreMesh(core_axis_name="c", subcore_axis_name="s"),
        scratch_types=[
            pltpu.VMEM((64,), jnp.int32),
            pltpu.VMEM((64, _DIM), _F32),
            pltpu.SemaphoreType.DMA,
        ],
    )


def _ffn_body(be_ref, xg_ref, w1_ref, w3_ref, w2_ref, y_ref, h_ref):
    j = pl.program_id(0)

    @pl.when(j < be_ref[127])
    def _active():
        _ffn_compute(xg_ref, w1_ref, w3_ref, w2_ref, y_ref, h_ref)


def _ffn_compute(xg_ref, w1_ref, w3_ref, w2_ref, y_ref, h_ref):
    x = xg_ref[...]
    for h in range(_H // _CH):
        w1c = w1_ref[0, pl.ds(h * _CH, _CH), :]
        w3c = w3_ref[0, pl.ds(h * _CH, _CH), :]
        a = lax.dot_general(x, w1c, (((1,), (1,)), ((), ())),
                            preferred_element_type=_F32)
        g = lax.dot_general(x, w3c, (((1,), (1,)), ((), ())),
                            preferred_element_type=_F32)
        h_ref[:, pl.ds(h * _CH, _CH)] = a * lax.logistic(a) * g
    y_ref[...] = lax.dot_general(h_ref[...], w2_ref[0], (((1,), (1,)), ((), ())),
                                 preferred_element_type=_F32)


def _ffn_call(bexp, xg, w1, w3, w2):
    grid_spec = pltpu.PrefetchScalarGridSpec(
        num_scalar_prefetch=1,
        grid=(_NB,),
        in_specs=[
            pl.BlockSpec((_BLK, _DIM), lambda j, be: (j, 0)),
            pl.BlockSpec((1, _H, _DIM), lambda j, be: (be[j], 0, 0)),
            pl.BlockSpec((1, _H, _DIM), lambda j, be: (be[j], 0, 0)),
            pl.BlockSpec((1, _DIM, _H), lambda j, be: (be[j], 0, 0),
                         pipeline_mode=pl.Buffered(buffer_count=1)),
        ],
        out_specs=pl.BlockSpec((_BLK, _DIM), lambda j, be: (j, 0)),
        scratch_shapes=[pltpu.VMEM((_BLK, _H), _F32)],
    )
    return pl.pallas_call(
        _ffn_body,
        grid_spec=grid_spec,
        out_shape=jax.ShapeDtypeStruct((_PADT, _DIM), _F32),
    )(bexp, xg, w1, w3, w2)


def _combine_body(y_hbm, d0_hbm, d1_hbm, o0_hbm, o1_hbm,
                  i0_v, i1_v, r0_v, r1_v, sem, sem2):
    wid = lax.axis_index("s") * 2 + lax.axis_index("c")
    per = _N // _NW
    rows = 32

    def chunk(c, carry):
        tb = wid * per + c * rows
        pltpu.sync_copy(d0_hbm.at[pl.ds(tb, rows)], i0_v)
        pltpu.sync_copy(d1_hbm.at[pl.ds(tb, rows)], i1_v)
        cp0 = pltpu.async_copy(y_hbm.at[i0_v], r0_v, sem)
        cp1 = pltpu.async_copy(y_hbm.at[i1_v], r1_v, sem2)
        cp0.wait()
        pltpu.sync_copy(r0_v, o0_hbm.at[pl.ds(tb, rows)])
        cp1.wait()
        pltpu.sync_copy(r1_v, o1_hbm.at[pl.ds(tb, rows)])
        return carry

    lax.fori_loop(0, per // rows, chunk, 0)


@functools.cache
def _combine_call():
    return pl.kernel(
        _combine_body,
        out_type=(jax.ShapeDtypeStruct((_N, _DIM), _F32),
                  jax.ShapeDtypeStruct((_N, _DIM), _F32)),
        mesh=plsc.VectorSubcoreMesh(core_axis_name="c", subcore_axis_name="s"),
        scratch_types=[
            pltpu.VMEM((32,), jnp.int32),
            pltpu.VMEM((32,), jnp.int32),
            pltpu.VMEM((32, _DIM), _F32),
            pltpu.VMEM((32, _DIM), _F32),
            pltpu.SemaphoreType.DMA,
            pltpu.SemaphoreType.DMA,
        ],
    )


def _mix_body(w0_ref, w1_ref, a_ref, b_ref, o_ref):
    o_ref[...] = w0_ref[...] * a_ref[...] + w1_ref[...] * b_ref[...]


def _mix_call(w0c, w1c, o0, o1):
    tm = 1024
    return pl.pallas_call(
        _mix_body,
        grid=(_N // tm,),
        in_specs=[
            pl.BlockSpec((tm, 1), lambda i: (i, 0)),
            pl.BlockSpec((tm, 1), lambda i: (i, 0)),
            pl.BlockSpec((tm, _DIM), lambda i: (i, 0)),
            pl.BlockSpec((tm, _DIM), lambda i: (i, 0)),
        ],
        out_specs=pl.BlockSpec((tm, _DIM), lambda i: (i, 0)),
        out_shape=jax.ShapeDtypeStruct((_N, _DIM), _F32),
    )(w0c, w1c, o0, o1)


def kernel(x, gate_W, w1, w2, w3):
    orig_shape = x.shape
    xf = x.reshape(_N, _DIM)
    d0, d1, w0c, w1c, be, aux = _gate_call(xf, gate_W)
    d0f = d0.reshape(_N)
    d1f = d1.reshape(_N)
    bexp = be.reshape(128)
    xg = _scatter_call()(xf, d0f, d1f)
    y = _ffn_call(bexp, xg, w1, w3, w2)
    o0, o1 = _combine_call()(y, d0f, d1f)
    out = _mix_call(w0c, w1c, o0, o1)
    return out.reshape(orig_shape), aux.reshape(())
